# baseline (speedup 1.0000x reference)
import functools

import jax
import jax.numpy as jnp
from jax import lax
from jax.experimental import pallas as pl
from jax.experimental.pallas import tpu as pltpu
from jax.experimental.pallas import tpu_sc as plsc

_N = 8388608
_NW = 32
_NBLK = _N // 128
_BLKW = _NBLK // _NW
_CBLK = 64
_NCHUNK = _BLKW // _CBLK
_NBINS = 1536
_LANES = 16


@functools.partial(
    pl.kernel,
    out_type=jax.ShapeDtypeStruct((_NW, _NBINS), jnp.float32),
    mesh=plsc.VectorSubcoreMesh(core_axis_name="c", subcore_axis_name="s"),
    scratch_types=[
        pltpu.VMEM((_CBLK, 2, 128), jnp.float32),
        pltpu.VMEM((_CBLK, 2, 128), jnp.float32),
        pltpu.VMEM((_CBLK, 1, 128), jnp.float32),
        pltpu.VMEM((_CBLK, 1, 128), jnp.float32),
        pltpu.VMEM((768 * _LANES,), jnp.float32),
        pltpu.VMEM((768 * _LANES,), jnp.float32),
        pltpu.VMEM((_NBINS,), jnp.float32),
        pltpu.SemaphoreType.DMA,
        pltpu.SemaphoreType.DMA,
        pltpu.SemaphoreType.DMA,
        pltpu.SemaphoreType.DMA,
    ],
    compiler_params=pltpu.CompilerParams(needs_layout_passes=False),
)
def _sc_hist(ev_hbm, out_hbm, bufxy0, bufxy1, bufp0, bufp1, histc, hists,
             part, semxy0, semxy1, semp0, semp1):
    wid = lax.axis_index("s") * 2 + lax.axis_index("c")
    iota = lax.iota(jnp.int32, _LANES)
    lane_base = iota * 768
    zeros = jnp.zeros((_LANES,), jnp.float32)
    ones = jnp.full((_LANES,), 1.0, jnp.float32)

    @plsc.parallel_loop(0, 768 * _LANES // _LANES)
    def _(i):
        histc[pl.ds(i * _LANES, _LANES)] = zeros
        hists[pl.ds(i * _LANES, _LANES)] = zeros

    bufxy = (bufxy0, bufxy1)
    bufp = (bufp0, bufp1)
    semxy = (semxy0, semxy1)
    semp = (semp0, semp1)
    base_blk = wid * _BLKW

    def make_copies(k, slot):
        b0 = base_blk + k * _CBLK
        return (
            pltpu.make_async_copy(
                ev_hbm.at[pl.ds(b0, _CBLK), pl.ds(0, 2), :],
                bufxy[slot], semxy[slot]),
            pltpu.make_async_copy(
                ev_hbm.at[pl.ds(b0, _CBLK), pl.ds(3, 1), :],
                bufp[slot], semp[slot]),
        )

    def start(k, slot):
        cxy, cp = make_copies(k, slot)
        cxy.start()
        cp.start()

    def wait(k, slot):
        cxy, cp = make_copies(k, slot)
        cxy.wait()
        cp.wait()

    def process(bxy, bp):
        @plsc.parallel_loop(0, _CBLK, unroll=2)
        def _(bb):
            for l in range(8):
                xv = bxy[bb, 0, pl.ds(l * _LANES, _LANES)]
                yv = bxy[bb, 1, pl.ds(l * _LANES, _LANES)]
                pv = bp[bb, 0, pl.ds(l * _LANES, _LANES)]
                xb = jnp.minimum((xv * jnp.float32(0.05)).astype(jnp.int32), 31)
                yb = jnp.minimum((yv * jnp.float32(0.05)).astype(jnp.int32), 23)
                idx = lane_base + (yb * 32 + xb)
                plsc.addupdate_scatter(histc, [idx], ones)
                plsc.addupdate_scatter(hists, [idx], pv)

    start(0, 0)
    start(1, 1)

    def cbody(g, c):
        k = g * 2
        for slot in (0, 1):
            wait(k + slot, slot)
            process(bufxy[slot], bufp[slot])

            @pl.when(k + slot + 2 < _NCHUNK)
            def _():
                start(k + slot + 2, slot)

        return c

    lax.fori_loop(0, _NCHUNK // 2, cbody, 0)

    @plsc.parallel_loop(0, 768 // _LANES)
    def _(j):
        accc = zeros
        accs = zeros
        for l in range(_LANES):
            accc = accc + histc[pl.ds(l * 768 + j * _LANES, _LANES)]
            accs = accs + hists[pl.ds(l * 768 + j * _LANES, _LANES)]
        part[pl.ds(j * _LANES, _LANES)] = accc
        part[pl.ds(768 + j * _LANES, _LANES)] = accs
    pltpu.sync_copy(part, out_hbm.at[wid])


def _finish_body(parts_ref, o_ref):
    parts = parts_ref[...]
    sums = jnp.sum(parts, axis=0, keepdims=True)
    c = sums[:, :768]
    s = sums[:, 768:]
    pos = (c + s) * jnp.float32(0.5)
    neg = (c - s) * jnp.float32(0.5)
    hist = jnp.concatenate([pos, neg], axis=1)
    total = jnp.sum(c)
    o_ref[...] = jnp.where(total > 0.0, hist / total, hist)


def kernel(events):
    ev_blocks = events.reshape(_NBLK, 128, 4).transpose(0, 2, 1)
    parts = _sc_hist(ev_blocks)
    out = pl.pallas_call(
        _finish_body,
        out_shape=jax.ShapeDtypeStruct((1, _NBINS), jnp.float32),
    )(parts)
    return out.reshape(2, 24, 32)


# speedup vs baseline: 1.0926x; 1.0926x over previous
import functools

import jax
import jax.numpy as jnp
from jax import lax
from jax.experimental import pallas as pl
from jax.experimental.pallas import tpu as pltpu
from jax.experimental.pallas import tpu_sc as plsc

_N = 8388608
_NW = 32
_NBLK = _N // 128
_BLKW = _NBLK // _NW
_CBLK = 128
_NCHUNK = _BLKW // _CBLK
_NBINS = 1536
_LANES = 16


@functools.partial(
    pl.kernel,
    out_type=jax.ShapeDtypeStruct((_NW, _NBINS), jnp.float32),
    mesh=plsc.VectorSubcoreMesh(core_axis_name="c", subcore_axis_name="s"),
    scratch_types=[
        pltpu.VMEM((_CBLK, 2, 128), jnp.float32),
        pltpu.VMEM((_CBLK, 2, 128), jnp.float32),
        pltpu.VMEM((_CBLK, 1, 128), jnp.float32),
        pltpu.VMEM((_CBLK, 1, 128), jnp.float32),
        pltpu.VMEM((_NBINS * _LANES,), jnp.float32),
        pltpu.VMEM((_NBINS,), jnp.float32),
        pltpu.SemaphoreType.DMA,
        pltpu.SemaphoreType.DMA,
        pltpu.SemaphoreType.DMA,
        pltpu.SemaphoreType.DMA,
    ],
    compiler_params=pltpu.CompilerParams(needs_layout_passes=False),
)
def _sc_hist(ev_hbm, out_hbm, bufxy0, bufxy1, bufp0, bufp1, hist, part,
             semxy0, semxy1, semp0, semp1):
    wid = lax.axis_index("s") * 2 + lax.axis_index("c")
    iota = lax.iota(jnp.int32, _LANES)
    lane_base = iota * _NBINS
    lane_base_neg = lane_base + 768
    zeros = jnp.zeros((_LANES,), jnp.float32)
    ones = jnp.full((_LANES,), 1.0, jnp.float32)

    @plsc.parallel_loop(0, _NBINS * _LANES // _LANES)
    def _(i):
        hist[pl.ds(i * _LANES, _LANES)] = zeros

    bufxy = (bufxy0, bufxy1)
    bufp = (bufp0, bufp1)
    semxy = (semxy0, semxy1)
    semp = (semp0, semp1)
    base_blk = wid * _BLKW

    def make_copies(k, slot):
        b0 = base_blk + k * _CBLK
        return (
            pltpu.make_async_copy(
                ev_hbm.at[pl.ds(b0, _CBLK), pl.ds(0, 2), :],
                bufxy[slot], semxy[slot]),
            pltpu.make_async_copy(
                ev_hbm.at[pl.ds(b0, _CBLK), pl.ds(3, 1), :],
                bufp[slot], semp[slot]),
        )

    def start(k, slot):
        cxy, cp = make_copies(k, slot)
        cxy.start()
        cp.start()

    def wait(k, slot):
        cxy, cp = make_copies(k, slot)
        cxy.wait()
        cp.wait()

    def process(bxy, bp):
        @plsc.parallel_loop(0, _CBLK, unroll=2)
        def _(bb):
            for l in range(8):
                xv = bxy[bb, 0, pl.ds(l * _LANES, _LANES)]
                yv = bxy[bb, 1, pl.ds(l * _LANES, _LANES)]
                pv = bp[bb, 0, pl.ds(l * _LANES, _LANES)]
                xb = jnp.minimum((xv * jnp.float32(0.05)).astype(jnp.int32), 31)
                yb = jnp.minimum((yv * jnp.float32(0.05)).astype(jnp.int32), 23)
                base = jnp.where(pv > 0.0, lane_base, lane_base_neg)
                idx = base + (yb * 32 + xb)
                plsc.addupdate_scatter(hist, [idx], ones)

    start(0, 0)
    start(1, 1)

    def cbody(g, c):
        k = g * 2
        for slot in (0, 1):
            wait(k + slot, slot)
            process(bufxy[slot], bufp[slot])

            @pl.when(k + slot + 2 < _NCHUNK)
            def _():
                start(k + slot + 2, slot)

        return c

    lax.fori_loop(0, _NCHUNK // 2, cbody, 0)

    @plsc.parallel_loop(0, _NBINS // _LANES)
    def _(j):
        acc = zeros
        for l in range(_LANES):
            acc = acc + hist[pl.ds(l * _NBINS + j * _LANES, _LANES)]
        part[pl.ds(j * _LANES, _LANES)] = acc
    pltpu.sync_copy(part, out_hbm.at[wid])


def _finish_body(parts_ref, o_ref):
    parts = parts_ref[...]
    hist = jnp.sum(parts, axis=0, keepdims=True)
    total = jnp.sum(hist)
    o_ref[...] = jnp.where(total > 0.0, hist / total, hist)


def kernel(events):
    ev_blocks = events.reshape(_NBLK, 128, 4).transpose(0, 2, 1)
    parts = _sc_hist(ev_blocks)
    out = pl.pallas_call(
        _finish_body,
        out_shape=jax.ShapeDtypeStruct((1, _NBINS), jnp.float32),
    )(parts)
    return out.reshape(2, 24, 32)
